# Initial kernel scaffold; baseline (speedup 1.0000x reference)
import jax
import jax.numpy as jnp
from jax.experimental import pallas as pl


def kernel(doc_idx, end_idx, match_data, embed):
    raise NotImplementedError("write your pallas kernel here")


# speedup vs baseline: 1.3292x; 1.3292x over previous
import functools

import jax
import jax.numpy as jnp
from jax import lax
from jax.experimental import pallas as pl
from jax.experimental.pallas import tpu as pltpu
from jax.experimental.pallas import tpu_sc as plsc

_B = 16
_S = 4096
_M = 32768
_K = 65536

_NC = 2
_NS = 16
_NW = _NC * _NS
_SLOTS = _B * _S
_SPW = _SLOTS // _NW
_LANES = 16


def _sc_body(doc_hbm, end_hbm, md_hbm, emb_hbm, out_hbm,
             doc_v, end_v, win_v, idx_v, md_v, emb_v, out_v, sem):
    wid = lax.axis_index("s") * _NC + lax.axis_index("c")
    base = wid * _SPW

    pltpu.sync_copy(doc_hbm, doc_v)
    pltpu.sync_copy(end_hbm, end_v)

    neg1 = jnp.full((_LANES,), -1, jnp.int32)

    def init_step(j, _):
        win_v[pl.ds(j * _LANES, _LANES)] = neg1
        return _

    lax.fori_loop(0, _SPW // _LANES, init_step, 0, unroll=4)

    iota = lax.iota(jnp.int32, _LANES)

    def scan_step(g, _):
        m0 = g * _LANES
        d = doc_v[pl.ds(m0, _LANES)]
        e = end_v[pl.ds(m0, _LANES)]
        flat = d * _S + e
        loc = flat - base
        inr = (loc >= 0) & (loc < _SPW)
        loc = jnp.where(inr, loc, 0)
        mvec = m0 + iota
        plsc.store_scatter(win_v, [loc], mvec, mask=inr)

        def cond(wb):
            return jnp.any(inr & (wb < mvec))

        def body(wb):
            pend = inr & (wb < mvec)
            plsc.store_scatter(win_v, [loc], mvec, mask=pend)
            return plsc.load_gather(win_v, [loc])

        wb0 = plsc.load_gather(win_v, [loc])
        lax.while_loop(cond, body, wb0)
        return _

    lax.fori_loop(0, _M // _LANES, scan_step, 0)

    def idx_step(j, _):
        w = win_v[pl.ds(j * _LANES, _LANES)]
        fallback = (base + j * _LANES + iota) & (_M - 1)
        idx_v[pl.ds(j * _LANES, _LANES)] = jnp.where(w >= 0, w, fallback)
        return _

    lax.fori_loop(0, _SPW // _LANES, idx_step, 0, unroll=4)

    pltpu.async_copy(md_hbm.at[idx_v], md_v, sem).wait()
    pltpu.async_copy(emb_hbm.at[md_v], emb_v, sem).wait()

    zero = jnp.zeros((_LANES,), jnp.float32)

    def out_step(j, _):
        w = win_v[pl.ds(j * _LANES, _LANES)]
        v = emb_v[pl.ds(j * _LANES, _LANES)]
        out_v[pl.ds(j * _LANES, _LANES)] = jnp.where(w >= 0, v, zero)
        return _

    lax.fori_loop(0, _SPW // _LANES, out_step, 0, unroll=4)

    pltpu.sync_copy(out_v, out_hbm.at[pl.ds(base, _SPW)])


_sc_call = functools.partial(
    pl.kernel,
    out_type=jax.ShapeDtypeStruct((_SLOTS,), jnp.float32),
    mesh=plsc.VectorSubcoreMesh(core_axis_name="c", subcore_axis_name="s"),
    compiler_params=pltpu.CompilerParams(needs_layout_passes=False),
    scratch_types=[
        pltpu.VMEM((_M,), jnp.int32),
        pltpu.VMEM((_M,), jnp.int32),
        pltpu.VMEM((_SPW,), jnp.int32),
        pltpu.VMEM((_SPW,), jnp.int32),
        pltpu.VMEM((_SPW,), jnp.int32),
        pltpu.VMEM((_SPW,), jnp.float32),
        pltpu.VMEM((_SPW,), jnp.float32),
        pltpu.SemaphoreType.DMA,
    ],
)(_sc_body)


def kernel(doc_idx, end_idx, match_data, embed):
    doc = doc_idx.astype(jnp.int32)
    end = end_idx.astype(jnp.int32)
    md = match_data.astype(jnp.int32)
    flat = _sc_call(doc, end, md, embed)
    return flat.reshape(_B, _S)


# speedup vs baseline: 2.1257x; 1.5993x over previous
import functools

import jax
import jax.numpy as jnp
from jax import lax
from jax.experimental import pallas as pl
from jax.experimental.pallas import tpu as pltpu
from jax.experimental.pallas import tpu_sc as plsc

_B = 16
_S = 4096
_M = 32768
_K = 65536

_NC = 2
_NS = 16
_NW = _NC * _NS
_SLOTS = _B * _S
_SPW = _SLOTS // _NW
_LANES = 16


def _sc_body(doc_hbm, end_hbm, md_hbm, emb_hbm, out_hbm,
             doc_v, end_v, win_v, cbuf_v, cnt_v, idx_v, md_v, emb_v, out_v,
             sem):
    wid = lax.axis_index("s") * _NC + lax.axis_index("c")
    base = wid * _SPW

    pltpu.sync_copy(doc_hbm, doc_v)
    pltpu.sync_copy(end_hbm, end_v)

    iota = lax.iota(jnp.int32, _LANES)
    neg1 = jnp.full((_LANES,), -1, jnp.int32)
    wid_splat = jnp.full((_LANES,), 0, jnp.int32) + wid

    def init_step(j, _):
        win_v[pl.ds(j * _LANES, _LANES)] = neg1
        return _

    lax.fori_loop(0, _SPW // _LANES, init_step, 0, unroll=8)

    def scan_step(g, carry):
        cursor, mvec = carry
        m0 = g * _LANES
        d = doc_v[pl.ds(m0, _LANES)]
        e = end_v[pl.ds(m0, _LANES)]
        flat = (d << 12) | e
        gk = (flat << 15) | mvec
        msk = (gk >> 26) == wid_splat
        pos = cursor + plsc.cumsum(msk.astype(jnp.int32))
        plsc.store_scatter(cbuf_v, [pos], gk, mask=msk)
        cursor = cursor + plsc.all_reduce_population_count(msk)
        return cursor, mvec + 16

    cursor, _ = lax.fori_loop(
        0, _M // _LANES, scan_step,
        (neg1, iota), unroll=4)

    cnt_splat = cursor + 1
    plsc.store_scatter(cbuf_v, [cnt_splat + iota], neg1)
    cnt = cnt_splat[0]

    rot_key = (iota + (_LANES - 1)) & (_LANES - 1)
    last_lane = iota == (_LANES - 1)

    def resolve_step(g, _):
        gk = cbuf_v[pl.ds(g * _LANES, _LANES)]
        (gk,) = lax.sort((gk,), dimension=0, num_keys=1)
        q = gk >> 15
        nxt = plsc.sort_key_val(rot_key, q)[1]
        valid = (gk >> 26) == wid_splat
        winner = valid & (last_lane | (q != nxt))
        loc = jnp.where(winner, q - (jnp.full((_LANES,), 0, jnp.int32) + base), 0)
        plsc.store_scatter(win_v, [loc], gk & (_M - 1), mask=winner)
        return _

    lax.fori_loop(0, (cnt + _LANES - 1) // _LANES, resolve_step, 0)

    def idx_step(j, _):
        w = win_v[pl.ds(j * _LANES, _LANES)]
        fallback = (base + j * _LANES + iota) & (_M - 1)
        idx_v[pl.ds(j * _LANES, _LANES)] = jnp.where(w >= 0, w, fallback)
        return _

    lax.fori_loop(0, _SPW // _LANES, idx_step, 0, unroll=8)

    pltpu.async_copy(md_hbm.at[idx_v], md_v, sem).wait()
    pltpu.async_copy(emb_hbm.at[md_v], emb_v, sem).wait()

    zero = jnp.zeros((_LANES,), jnp.float32)

    def out_step(j, _):
        w = win_v[pl.ds(j * _LANES, _LANES)]
        v = emb_v[pl.ds(j * _LANES, _LANES)]
        out_v[pl.ds(j * _LANES, _LANES)] = jnp.where(w >= 0, v, zero)
        return _

    lax.fori_loop(0, _SPW // _LANES, out_step, 0, unroll=8)

    pltpu.sync_copy(out_v, out_hbm.at[pl.ds(base, _SPW)])


_sc_call = functools.partial(
    pl.kernel,
    out_type=jax.ShapeDtypeStruct((_SLOTS,), jnp.float32),
    mesh=plsc.VectorSubcoreMesh(core_axis_name="c", subcore_axis_name="s"),
    compiler_params=pltpu.CompilerParams(needs_layout_passes=False),
    scratch_types=[
        pltpu.VMEM((_M,), jnp.int32),
        pltpu.VMEM((_M,), jnp.int32),
        pltpu.VMEM((_SPW,), jnp.int32),
        pltpu.VMEM((_M + _LANES,), jnp.int32),
        pltpu.VMEM((_LANES,), jnp.int32),
        pltpu.VMEM((_SPW,), jnp.int32),
        pltpu.VMEM((_SPW,), jnp.int32),
        pltpu.VMEM((_SPW,), jnp.float32),
        pltpu.VMEM((_SPW,), jnp.float32),
        pltpu.SemaphoreType.DMA,
    ],
)(_sc_body)


def kernel(doc_idx, end_idx, match_data, embed):
    doc = doc_idx.astype(jnp.int32)
    end = end_idx.astype(jnp.int32)
    md = match_data.astype(jnp.int32)
    flat = _sc_call(doc, end, md, embed)
    return flat.reshape(_B, _S)


# speedup vs baseline: 2.3205x; 1.0917x over previous
import functools

import jax
import jax.numpy as jnp
from jax import lax
from jax.experimental import pallas as pl
from jax.experimental.pallas import tpu as pltpu
from jax.experimental.pallas import tpu_sc as plsc

_B = 16
_S = 4096
_M = 32768
_K = 65536

_NC = 2
_NS = 16
_NW = _NC * _NS
_SLOTS = _B * _S
_SPW = _SLOTS // _NW
_LANES = 16


def _sc_body(doc_hbm, end_hbm, md_hbm, emb_hbm, out_hbm,
             doc_v, end_v, win_v, cbuf_v, cnt_v, idx_v, md_v, emb_v, out_v,
             sem):
    wid = lax.axis_index("s") * _NC + lax.axis_index("c")
    base = wid * _SPW

    pltpu.sync_copy(doc_hbm, doc_v)
    pltpu.sync_copy(end_hbm, end_v)

    iota = lax.iota(jnp.int32, _LANES)
    neg1 = jnp.full((_LANES,), -1, jnp.int32)
    wid_splat = jnp.full((_LANES,), 0, jnp.int32) + wid

    def init_step(j, _):
        win_v[pl.ds(j * _LANES, _LANES)] = neg1
        return _

    lax.fori_loop(0, _SPW // _LANES, init_step, 0, unroll=8)

    def scan_step(g, carry):
        cnt, mvec = carry
        m0 = g * _LANES
        d = doc_v[pl.ds(m0, _LANES)]
        e = end_v[pl.ds(m0, _LANES)]
        flat = (d << 12) | e
        gk = (flat << 15) | mvec
        msk = (gk >> 26) == wid_splat
        plsc.store_compressed(cbuf_v.at[pl.ds(cnt, _LANES)], gk, mask=msk)
        cnt = cnt + plsc.all_reduce_population_count(msk)[0]
        return cnt, mvec + 16

    cnt, _ = lax.fori_loop(
        0, _M // _LANES, scan_step,
        (jnp.int32(0), iota), unroll=4)

    plsc.store_scatter(cbuf_v, [cnt + iota], neg1)

    rot_key = (iota + (_LANES - 1)) & (_LANES - 1)
    last_lane = iota == (_LANES - 1)

    def resolve_step(g, _):
        gk = cbuf_v[pl.ds(g * _LANES, _LANES)]
        (gk,) = lax.sort((gk,), dimension=0, num_keys=1)
        q = gk >> 15
        nxt = plsc.sort_key_val(rot_key, q)[1]
        valid = (gk >> 26) == wid_splat
        winner = valid & (last_lane | (q != nxt))
        loc = jnp.where(winner, q - (jnp.full((_LANES,), 0, jnp.int32) + base), 0)
        plsc.store_scatter(win_v, [loc], gk & (_M - 1), mask=winner)
        return _

    lax.fori_loop(0, (cnt + _LANES - 1) // _LANES, resolve_step, 0)

    def idx_step(j, _):
        w = win_v[pl.ds(j * _LANES, _LANES)]
        fallback = (base + j * _LANES + iota) & (_M - 1)
        idx_v[pl.ds(j * _LANES, _LANES)] = jnp.where(w >= 0, w, fallback)
        return _

    lax.fori_loop(0, _SPW // _LANES, idx_step, 0, unroll=8)

    pltpu.async_copy(md_hbm.at[idx_v], md_v, sem).wait()
    pltpu.async_copy(emb_hbm.at[md_v], emb_v, sem).wait()

    zero = jnp.zeros((_LANES,), jnp.float32)

    def out_step(j, _):
        w = win_v[pl.ds(j * _LANES, _LANES)]
        v = emb_v[pl.ds(j * _LANES, _LANES)]
        out_v[pl.ds(j * _LANES, _LANES)] = jnp.where(w >= 0, v, zero)
        return _

    lax.fori_loop(0, _SPW // _LANES, out_step, 0, unroll=8)

    pltpu.sync_copy(out_v, out_hbm.at[pl.ds(base, _SPW)])


_sc_call = functools.partial(
    pl.kernel,
    out_type=jax.ShapeDtypeStruct((_SLOTS,), jnp.float32),
    mesh=plsc.VectorSubcoreMesh(core_axis_name="c", subcore_axis_name="s"),
    compiler_params=pltpu.CompilerParams(needs_layout_passes=False),
    scratch_types=[
        pltpu.VMEM((_M,), jnp.int32),
        pltpu.VMEM((_M,), jnp.int32),
        pltpu.VMEM((_SPW,), jnp.int32),
        pltpu.VMEM((_M + _LANES,), jnp.int32),
        pltpu.VMEM((_LANES,), jnp.int32),
        pltpu.VMEM((_SPW,), jnp.int32),
        pltpu.VMEM((_SPW,), jnp.int32),
        pltpu.VMEM((_SPW,), jnp.float32),
        pltpu.VMEM((_SPW,), jnp.float32),
        pltpu.SemaphoreType.DMA,
    ],
)(_sc_body)


def kernel(doc_idx, end_idx, match_data, embed):
    doc = doc_idx.astype(jnp.int32)
    end = end_idx.astype(jnp.int32)
    md = match_data.astype(jnp.int32)
    flat = _sc_call(doc, end, md, embed)
    return flat.reshape(_B, _S)


# speedup vs baseline: 3.3700x; 1.4523x over previous
import functools

import jax
import jax.numpy as jnp
from jax import lax
from jax.experimental import pallas as pl
from jax.experimental.pallas import tpu as pltpu
from jax.experimental.pallas import tpu_sc as plsc

_B = 16
_S = 4096
_M = 32768
_K = 65536

_NC = 2
_NS = 16
_NW = _NC * _NS
_SLOTS = _B * _S
_SPW = _SLOTS // _NW
_LANES = 16


def _sc_body(doc_hbm, end_hbm, md_hbm, emb_hbm, out_hbm,
             doc_v, end_v, win_v, cbuf_v, cnt_v, idx_v, md_v, emb_v, out_v,
             sem):
    wid = lax.axis_index("s") * _NC + lax.axis_index("c")
    base = wid * _SPW

    pltpu.sync_copy(doc_hbm, doc_v)
    pltpu.sync_copy(end_hbm, end_v)

    iota = lax.iota(jnp.int32, _LANES)
    neg1 = jnp.full((_LANES,), -1, jnp.int32)
    wid_splat = jnp.full((_LANES,), 0, jnp.int32) + wid

    @plsc.parallel_loop(0, _SPW // _LANES, unroll=8)
    def init_step(j):
        win_v[pl.ds(j * _LANES, _LANES)] = neg1

    @plsc.parallel_loop(0, _M // _LANES, unroll=4,
                        carry=(jnp.int32(0), iota))
    def scan_step(g, carry):
        cnt, mvec = carry
        m0 = g * _LANES
        d = doc_v[pl.ds(m0, _LANES)]
        e = end_v[pl.ds(m0, _LANES)]
        flat = (d << 12) | e
        gk = (flat << 15) | mvec
        msk = (gk >> 26) == wid_splat
        plsc.store_compressed(cbuf_v.at[pl.ds(cnt, _LANES)], gk, mask=msk)
        cnt = cnt + plsc.all_reduce_population_count(msk)[0]
        return cnt, mvec + 16

    cnt, _ = scan_step

    plsc.store_scatter(cbuf_v, [cnt + iota], neg1)

    rot_key = (iota + (_LANES - 1)) & (_LANES - 1)
    last_lane = iota == (_LANES - 1)

    def resolve_step(g, _):
        gk = cbuf_v[pl.ds(g * _LANES, _LANES)]
        (gk,) = lax.sort((gk,), dimension=0, num_keys=1)
        q = gk >> 15
        nxt = plsc.sort_key_val(rot_key, q)[1]
        valid = (gk >> 26) == wid_splat
        winner = valid & (last_lane | (q != nxt))
        loc = jnp.where(winner, q - (jnp.full((_LANES,), 0, jnp.int32) + base), 0)
        plsc.store_scatter(win_v, [loc], gk & (_M - 1), mask=winner)
        return _

    lax.fori_loop(0, (cnt + _LANES - 1) // _LANES, resolve_step, 0)

    @plsc.parallel_loop(0, _SPW // _LANES, unroll=8)
    def idx_step(j):
        w = win_v[pl.ds(j * _LANES, _LANES)]
        fallback = (base + j * _LANES + iota) & (_M - 1)
        idx_v[pl.ds(j * _LANES, _LANES)] = jnp.where(w >= 0, w, fallback)

    pltpu.async_copy(md_hbm.at[idx_v], md_v, sem).wait()
    pltpu.async_copy(emb_hbm.at[md_v], emb_v, sem).wait()

    zero = jnp.zeros((_LANES,), jnp.float32)

    @plsc.parallel_loop(0, _SPW // _LANES, unroll=8)
    def out_step(j):
        w = win_v[pl.ds(j * _LANES, _LANES)]
        v = emb_v[pl.ds(j * _LANES, _LANES)]
        out_v[pl.ds(j * _LANES, _LANES)] = jnp.where(w >= 0, v, zero)

    pltpu.sync_copy(out_v, out_hbm.at[pl.ds(base, _SPW)])


_sc_call = functools.partial(
    pl.kernel,
    out_type=jax.ShapeDtypeStruct((_SLOTS,), jnp.float32),
    mesh=plsc.VectorSubcoreMesh(core_axis_name="c", subcore_axis_name="s"),
    compiler_params=pltpu.CompilerParams(needs_layout_passes=False),
    scratch_types=[
        pltpu.VMEM((_M,), jnp.int32),
        pltpu.VMEM((_M,), jnp.int32),
        pltpu.VMEM((_SPW,), jnp.int32),
        pltpu.VMEM((_M + _LANES,), jnp.int32),
        pltpu.VMEM((_LANES,), jnp.int32),
        pltpu.VMEM((_SPW,), jnp.int32),
        pltpu.VMEM((_SPW,), jnp.int32),
        pltpu.VMEM((_SPW,), jnp.float32),
        pltpu.VMEM((_SPW,), jnp.float32),
        pltpu.SemaphoreType.DMA,
    ],
)(_sc_body)


def kernel(doc_idx, end_idx, match_data, embed):
    doc = doc_idx.astype(jnp.int32)
    end = end_idx.astype(jnp.int32)
    md = match_data.astype(jnp.int32)
    flat = _sc_call(doc, end, md, embed)
    return flat.reshape(_B, _S)


# speedup vs baseline: 3.4022x; 1.0095x over previous
import functools

import jax
import jax.numpy as jnp
from jax import lax
from jax.experimental import pallas as pl
from jax.experimental.pallas import tpu as pltpu
from jax.experimental.pallas import tpu_sc as plsc

_B = 16
_S = 4096
_M = 32768
_K = 65536

_NC = 2
_NS = 16
_NW = _NC * _NS
_SLOTS = _B * _S
_SPW = _SLOTS // _NW
_LANES = 16


def _sc_body(doc_hbm, end_hbm, md_hbm, emb_hbm, out_hbm,
             doc_v, end_v, win_v, cbuf_v, cnt_v, idx_v, md_v, emb_v, out_v,
             sem):
    wid = lax.axis_index("s") * _NC + lax.axis_index("c")
    base = wid * _SPW

    doc_dma = pltpu.async_copy(doc_hbm, doc_v, sem)
    end_dma = pltpu.async_copy(end_hbm, end_v, sem)

    iota = lax.iota(jnp.int32, _LANES)
    neg1 = jnp.full((_LANES,), -1, jnp.int32)
    wid_splat = jnp.full((_LANES,), 0, jnp.int32) + wid

    @plsc.parallel_loop(0, _SPW // _LANES, unroll=8)
    def init_step(j):
        win_v[pl.ds(j * _LANES, _LANES)] = neg1

    doc_dma.wait()
    end_dma.wait()

    @plsc.parallel_loop(0, _M // _LANES, unroll=8,
                        carry=(jnp.int32(0), iota))
    def scan_step(g, carry):
        cnt, mvec = carry
        m0 = g * _LANES
        d = doc_v[pl.ds(m0, _LANES)]
        e = end_v[pl.ds(m0, _LANES)]
        flat = (d << 12) | e
        gk = (flat << 15) | mvec
        msk = (gk >> 26) == wid_splat
        plsc.store_compressed(cbuf_v.at[pl.ds(cnt, _LANES)], gk, mask=msk)
        cnt = cnt + plsc.all_reduce_population_count(msk)[0]
        return cnt, mvec + 16

    cnt, _ = scan_step

    plsc.store_scatter(cbuf_v, [cnt + iota], neg1)

    rot_key = (iota + (_LANES - 1)) & (_LANES - 1)
    last_lane = iota == (_LANES - 1)

    def resolve_step(g, _):
        gk = cbuf_v[pl.ds(g * _LANES, _LANES)]
        (gk,) = lax.sort((gk,), dimension=0, num_keys=1)
        q = gk >> 15
        nxt = plsc.sort_key_val(rot_key, q)[1]
        valid = (gk >> 26) == wid_splat
        winner = valid & (last_lane | (q != nxt))
        loc = jnp.where(winner, q - (jnp.full((_LANES,), 0, jnp.int32) + base), 0)
        plsc.store_scatter(win_v, [loc], gk & (_M - 1), mask=winner)
        return _

    lax.fori_loop(0, (cnt + _LANES - 1) // _LANES, resolve_step, 0)

    @plsc.parallel_loop(0, _SPW // _LANES, unroll=8)
    def idx_step(j):
        w = win_v[pl.ds(j * _LANES, _LANES)]
        fallback = (base + j * _LANES + iota) & (_M - 1)
        idx_v[pl.ds(j * _LANES, _LANES)] = jnp.where(w >= 0, w, fallback)

    pltpu.async_copy(md_hbm.at[idx_v], md_v, sem).wait()
    pltpu.async_copy(emb_hbm.at[md_v], emb_v, sem).wait()

    zero = jnp.zeros((_LANES,), jnp.float32)

    @plsc.parallel_loop(0, _SPW // _LANES, unroll=8)
    def out_step(j):
        w = win_v[pl.ds(j * _LANES, _LANES)]
        v = emb_v[pl.ds(j * _LANES, _LANES)]
        out_v[pl.ds(j * _LANES, _LANES)] = jnp.where(w >= 0, v, zero)

    pltpu.sync_copy(out_v, out_hbm.at[pl.ds(base, _SPW)])


_sc_call = functools.partial(
    pl.kernel,
    out_type=jax.ShapeDtypeStruct((_SLOTS,), jnp.float32),
    mesh=plsc.VectorSubcoreMesh(core_axis_name="c", subcore_axis_name="s"),
    compiler_params=pltpu.CompilerParams(needs_layout_passes=False),
    scratch_types=[
        pltpu.VMEM((_M,), jnp.int32),
        pltpu.VMEM((_M,), jnp.int32),
        pltpu.VMEM((_SPW,), jnp.int32),
        pltpu.VMEM((_M + _LANES,), jnp.int32),
        pltpu.VMEM((_LANES,), jnp.int32),
        pltpu.VMEM((_SPW,), jnp.int32),
        pltpu.VMEM((_SPW,), jnp.int32),
        pltpu.VMEM((_SPW,), jnp.float32),
        pltpu.VMEM((_SPW,), jnp.float32),
        pltpu.SemaphoreType.DMA,
    ],
)(_sc_body)


def kernel(doc_idx, end_idx, match_data, embed):
    doc = doc_idx.astype(jnp.int32)
    end = end_idx.astype(jnp.int32)
    md = match_data.astype(jnp.int32)
    flat = _sc_call(doc, end, md, embed)
    return flat.reshape(_B, _S)
